# Initial kernel scaffold; baseline (speedup 1.0000x reference)
import jax
import jax.numpy as jnp
from jax.experimental import pallas as pl


def kernel(x, weight):
    raise NotImplementedError("write your pallas kernel here")


# speedup vs baseline: 1.5763x; 1.5763x over previous
import functools

import jax
import jax.numpy as jnp
from jax import lax
from jax.experimental import pallas as pl
from jax.experimental.pallas import tpu as pltpu
from jax.experimental.pallas import tpu_sc as plsc

NUM_FEAT = 1000000
HIDDEN_DIM = 32
BATCH = 16384
FIELDS = 26

_INFO = plsc.get_sparse_core_info()
_NC = _INFO.num_cores
_NS = _INFO.num_subcores
_NW = _NC * _NS

_B = BATCH * FIELDS
_BPW = _B // _NW
_CHUNK = 1024
_NCHUNK = _BPW // _CHUNK


def _make_kernel():
    mesh = plsc.VectorSubcoreMesh(core_axis_name="c", subcore_axis_name="s")

    @functools.partial(
        pl.kernel,
        mesh=mesh,
        out_type=jax.ShapeDtypeStruct((_B, HIDDEN_DIM), jnp.float32),
        compiler_params=pltpu.CompilerParams(use_tc_tiling_on_sc=False),
        scratch_types=[
            pltpu.VMEM((_BPW,), jnp.int32),
            pltpu.VMEM((_CHUNK, HIDDEN_DIM), jnp.float32),
            pltpu.VMEM((_CHUNK, HIDDEN_DIM), jnp.float32),
            pltpu.SemaphoreType.DMA,
            pltpu.SemaphoreType.DMA,
            pltpu.SemaphoreType.DMA,
        ],
    )
    def emb_kernel(idx_hbm, table_hbm, out_hbm,
                   idx_v, rows0, rows1, isem, gsem0, gsem1):
        wid = lax.axis_index("s") * _NC + lax.axis_index("c")
        base = wid * _BPW
        pltpu.async_copy(idx_hbm.at[wid], idx_v, isem).wait()

        bufs = (rows0, rows1)
        sems = (gsem0, gsem1)
        pend = pltpu.async_copy(
            table_hbm.at[idx_v.at[pl.ds(0, _CHUNK)]], bufs[0], sems[0])
        for c in range(_NCHUNK):
            nxt = None
            if c + 1 < _NCHUNK:
                nxt = pltpu.async_copy(
                    table_hbm.at[idx_v.at[pl.ds((c + 1) * _CHUNK, _CHUNK)]],
                    bufs[(c + 1) % 2], sems[(c + 1) % 2])
            pend.wait()
            pltpu.sync_copy(bufs[c % 2],
                            out_hbm.at[pl.ds(base + c * _CHUNK, _CHUNK)])
            pend = nxt

    return emb_kernel


_EMB = _make_kernel()


@jax.jit
def kernel(x, weight):
    idx = x.astype(jnp.int32).reshape(_NW, _BPW)
    out = _EMB(idx, weight)
    return out.reshape(BATCH, FIELDS, HIDDEN_DIM)
